# Initial kernel scaffold; baseline (speedup 1.0000x reference)
import jax
import jax.numpy as jnp
from jax.experimental import pallas as pl


def kernel(data, labels):
    raise NotImplementedError("write your pallas kernel here")


# speedup vs baseline: 49.9117x; 49.9117x over previous
import functools

import jax
import jax.numpy as jnp
from jax import lax
from jax.experimental import pallas as pl
from jax.experimental.pallas import tpu as pltpu
from jax.experimental.pallas import tpu_sc as plsc

MARGIN = 1.0
N_TOTAL = 16 * 3 * 512 * 512
NW = 32
L = 16
PER_W = N_TOTAL // NW
CHUNK = 8192
VECS = CHUNK // L
NCHUNK = PER_W // CHUNK
NB = 1024
HSZ = NB * L
B1_SHIFT = 22
B2_SHIFT = 12

_mesh = plsc.VectorSubcoreMesh(core_axis_name="c", subcore_axis_name="s")


def _worker_id():
    return lax.axis_index("s") * 2 + lax.axis_index("c")


def _keys_bins(d):
    bits = plsc.bitcast(d, jnp.uint32)
    isneg = bits >= jnp.uint32(0x80000000)
    key = jnp.where(isneg, ~bits, bits | jnp.uint32(0x80000000))
    bin1 = (key >> jnp.uint32(B1_SHIFT)).astype(jnp.int32)
    bin2 = ((key >> jnp.uint32(B2_SHIFT)) & jnp.uint32(NB - 1)).astype(jnp.int32)
    return bin1, bin2


def _zero_hist(hist):
    zero = jnp.zeros((L,), jnp.float32)

    def zbody(i, carry):
        hist[pl.ds(i * L, L)] = zero
        return carry

    lax.fori_loop(0, 4 * HSZ // L, zbody, 0)


@functools.partial(
    pl.kernel,
    out_type=(
        jax.ShapeDtypeStruct((NW, 4 * HSZ), jnp.float32),
        jax.ShapeDtypeStruct((NW, L), jnp.float32),
    ),
    mesh=_mesh,
    compiler_params=pltpu.CompilerParams(needs_layout_passes=False),
    scratch_types=(
        pltpu.VMEM((CHUNK,), jnp.float32),
        pltpu.VMEM((CHUNK,), jnp.int32),
        pltpu.VMEM((4 * HSZ,), jnp.float32),
        pltpu.VMEM((L,), jnp.float32),
    ),
)
def _pass1(data_hbm, labels_hbm, hist_out, loss_out, dbuf, lbuf, hist, lstage):
    wid = _worker_id()
    base = wid * PER_W
    _zero_hist(hist)
    lane = lax.iota(jnp.int32, L)
    ones = jnp.ones((L,), jnp.float32)

    def chunk_body(c, ls):
        start = base + c * CHUNK
        pltpu.sync_copy(data_hbm.at[pl.ds(start, CHUNK)], dbuf)
        pltpu.sync_copy(labels_hbm.at[pl.ds(start, CHUNK)], lbuf)

        def vec_body(v, ls):
            d = dbuf[pl.ds(v * L, L)]
            lab = lbuf[pl.ds(v * L, L)]
            pos = lab != 0
            hneg = jnp.logical_and(lab == 0, d <= MARGIN)
            bin1, _ = _keys_bins(d)
            idx = bin1 * L + lane
            dsq = d * d
            t = MARGIN - d
            tsq = t * t
            plsc.addupdate_scatter(hist, (idx,), ones, mask=pos)
            plsc.addupdate_scatter(hist, (idx + HSZ,), dsq, mask=pos)
            plsc.addupdate_scatter(hist, (idx + 2 * HSZ,), ones, mask=hneg)
            plsc.addupdate_scatter(hist, (idx + 3 * HSZ,), tsq, mask=hneg)
            tr = jnp.maximum(t, 0.0)
            return ls + jnp.where(pos, dsq, tr * tr)

        return lax.fori_loop(0, VECS, vec_body, ls)

    ls = lax.fori_loop(0, NCHUNK, chunk_body, jnp.zeros((L,), jnp.float32))
    lstage[...] = ls
    pltpu.sync_copy(hist, hist_out.at[wid])
    pltpu.sync_copy(lstage, loss_out.at[wid])


@functools.partial(
    pl.kernel,
    out_type=jax.ShapeDtypeStruct((NW, 4 * HSZ), jnp.float32),
    mesh=_mesh,
    compiler_params=pltpu.CompilerParams(needs_layout_passes=False),
    scratch_types=(
        pltpu.VMEM((CHUNK,), jnp.float32),
        pltpu.VMEM((CHUNK,), jnp.int32),
        pltpu.VMEM((4 * HSZ,), jnp.float32),
        pltpu.VMEM((2 * L,), jnp.int32),
    ),
)
def _pass2(data_hbm, labels_hbm, thr_hbm, hist_out, dbuf, lbuf, hist, tbuf):
    wid = _worker_id()
    base = wid * PER_W
    _zero_hist(hist)
    pltpu.sync_copy(thr_hbm, tbuf)
    thrp = tbuf[pl.ds(0, L)]
    thrn = tbuf[pl.ds(L, L)]
    lane = lax.iota(jnp.int32, L)
    ones = jnp.ones((L,), jnp.float32)

    def chunk_body(c, carry):
        start = base + c * CHUNK
        pltpu.sync_copy(data_hbm.at[pl.ds(start, CHUNK)], dbuf)
        pltpu.sync_copy(labels_hbm.at[pl.ds(start, CHUNK)], lbuf)

        def vec_body(v, carry):
            d = dbuf[pl.ds(v * L, L)]
            lab = lbuf[pl.ds(v * L, L)]
            pos = lab != 0
            hneg = jnp.logical_and(lab == 0, d <= MARGIN)
            bin1, bin2 = _keys_bins(d)
            in_p = jnp.logical_and(pos, bin1 == thrp)
            in_n = jnp.logical_and(hneg, bin1 == thrn)
            idx = bin2 * L + lane
            dsq = d * d
            t = MARGIN - d
            plsc.addupdate_scatter(hist, (idx,), ones, mask=in_p)
            plsc.addupdate_scatter(hist, (idx + HSZ,), dsq, mask=in_p)
            plsc.addupdate_scatter(hist, (idx + 2 * HSZ,), ones, mask=in_n)
            plsc.addupdate_scatter(hist, (idx + 3 * HSZ,), t * t, mask=in_n)
            return carry

        return lax.fori_loop(0, VECS, vec_body, carry)

    lax.fori_loop(0, NCHUNK, chunk_body, 0)
    pltpu.sync_copy(hist, hist_out.at[wid])


def _select_desc(cnt, ssum, m):
    suf_c = jnp.cumsum(cnt[::-1])[::-1]
    ok = suf_c >= m
    b = NB - 1 - jnp.argmax(ok[::-1])
    above_c = suf_c[b] - cnt[b]
    suf_s = jnp.cumsum(ssum[::-1])[::-1]
    above_s = suf_s[b] - ssum[b]
    return above_s, b, m - above_c


def _select_asc(cnt, ssum, m):
    pre_c = jnp.cumsum(cnt)
    ok = pre_c >= m
    b = jnp.argmax(ok)
    below_c = pre_c[b] - cnt[b]
    pre_s = jnp.cumsum(ssum)
    below_s = pre_s[b] - ssum[b]
    return below_s, b, m - below_c


def _frac_part(cnt_b, sum_b, r):
    safe = jnp.maximum(cnt_b, 1.0)
    return jnp.where(cnt_b > 0.0, (r / safe) * sum_b, 0.0)


def kernel(data, labels):
    d = data.reshape(-1)
    lab = labels.reshape(-1)

    hist1, loss_p = _pass1(d, lab)
    h1 = hist1.reshape(NW, 4, NB, L).sum(axis=(0, 3))
    pc, ps, nc, ns = h1[0], h1[1], h1[2], h1[3]
    n_pos = jnp.sum(pc)
    n_neg = jnp.sum(nc)
    m = jnp.minimum(n_pos, n_neg)

    ps_above, bp, need_p = _select_desc(pc, ps, m)
    ns_below, bn, need_n = _select_asc(nc, ns, m)

    thr = jnp.concatenate(
        [jnp.full((L,), bp, jnp.int32), jnp.full((L,), bn, jnp.int32)]
    )
    hist2 = _pass2(d, lab, thr)
    h2 = hist2.reshape(NW, 4, NB, L).sum(axis=(0, 3))
    pc2, ps2, nc2, ns2 = h2[0], h2[1], h2[2], h2[3]

    ps2_above, sp, r_p = _select_desc(pc2, ps2, need_p)
    ns2_below, sn, r_n = _select_asc(nc2, ns2, need_n)

    sum_pos = ps_above + ps2_above + _frac_part(pc2[sp], ps2[sp], r_p)
    sum_neg = ns_below + ns2_below + _frac_part(nc2[sn], ns2[sn], r_n)

    mined = (sum_pos + sum_neg) / (2.0 * m)
    loss_all = jnp.sum(loss_p) / jnp.float32(N_TOTAL)
    return jnp.where(n_neg == 0, loss_all, mined)


# speedup vs baseline: 66.0419x; 1.3232x over previous
import functools

import jax
import jax.numpy as jnp
from jax import lax
from jax.experimental import pallas as pl
from jax.experimental.pallas import tpu as pltpu
from jax.experimental.pallas import tpu_sc as plsc

MARGIN = 1.0
N_TOTAL = 16 * 3 * 512 * 512
NW = 32
L = 16
PER_W = N_TOTAL // NW
CHUNK = 8192
VECS = CHUNK // L
NCHUNK = PER_W // CHUNK
NGRP = NCHUNK // 2
UNROLL = 4
NB = 1024
HSZ = NB * L
B1_SHIFT = 22
B2_SHIFT = 12

_mesh = plsc.VectorSubcoreMesh(core_axis_name="c", subcore_axis_name="s")
_params = pltpu.CompilerParams(needs_layout_passes=False)


def _keys_bins(d, want2):
    bits = plsc.bitcast(d, jnp.int32)
    key = bits ^ ((bits >> 31) | jnp.int32(-(2**31)))
    keyu = plsc.bitcast(key, jnp.uint32)
    bin1 = (keyu >> jnp.uint32(B1_SHIFT)).astype(jnp.int32)
    bin2 = None
    if want2:
        bin2 = ((keyu >> jnp.uint32(B2_SHIFT)) & jnp.uint32(NB - 1)).astype(jnp.int32)
    return bin1, bin2


def _zero_hist(hist):
    zero = jnp.zeros((L,), jnp.float32)

    def zbody(i, carry):
        for u in range(8):
            hist[pl.ds((i * 8 + u) * L, L)] = zero
        return carry

    lax.fori_loop(0, 4 * HSZ // (8 * L), zbody, 0)


def _make_pass(refine):
    def body(data_hbm, labels_hbm, *rest):
        if refine:
            (thr_hbm, hist_out,
             d0, d1, l0, l1, hist, tbuf,
             sd0, sd1, sl0, sl1) = rest
        else:
            (hist_out, loss_out,
             d0, d1, l0, l1, hist, lstage,
             sd0, sd1, sl0, sl1) = rest
        dbufs, lbufs = (d0, d1), (l0, l1)
        dsems, lsems = (sd0, sd1), (sl0, sl1)

        wid = lax.axis_index("s") * 2 + lax.axis_index("c")
        base = wid * PER_W
        _zero_hist(hist)
        if refine:
            pltpu.sync_copy(thr_hbm, tbuf)
            thrp = tbuf[pl.ds(0, L)]
            thrn = tbuf[pl.ds(L, L)]
        lane = lax.iota(jnp.int32, L)
        ones = jnp.ones((L,), jnp.float32)

        def issue(c, b):
            start = base + c * CHUNK
            pltpu.async_copy(data_hbm.at[pl.ds(start, CHUNK)], dbufs[b], dsems[b])
            pltpu.async_copy(labels_hbm.at[pl.ds(start, CHUNK)], lbufs[b], lsems[b])

        def drain(b):
            pltpu.make_async_copy(
                data_hbm.at[pl.ds(0, CHUNK)], dbufs[b], dsems[b]).wait()
            pltpu.make_async_copy(
                labels_hbm.at[pl.ds(0, CHUNK)], lbufs[b], lsems[b]).wait()

        def consume(b, ls):
            dbuf, lbuf = dbufs[b], lbufs[b]

            def vec_body(v, ls):
                for u in range(UNROLL):
                    off = (v * UNROLL + u) * L
                    d = dbuf[pl.ds(off, L)]
                    lab = lbuf[pl.ds(off, L)]
                    pos = lab != 0
                    hneg = jnp.logical_and(lab == 0, d <= MARGIN)
                    bin1, bin2 = _keys_bins(d, refine)
                    dsq = d * d
                    t = MARGIN - d
                    tsq = t * t
                    if refine:
                        mp = jnp.logical_and(pos, bin1 == thrp)
                        mn = jnp.logical_and(hneg, bin1 == thrn)
                        idx = bin2 * L + lane
                    else:
                        mp, mn = pos, hneg
                        idx = bin1 * L + lane
                    plsc.addupdate_scatter(hist, (idx,), ones, mask=mp)
                    plsc.addupdate_scatter(hist, (idx + HSZ,), dsq, mask=mp)
                    plsc.addupdate_scatter(hist, (idx + 2 * HSZ,), ones, mask=mn)
                    plsc.addupdate_scatter(hist, (idx + 3 * HSZ,), tsq, mask=mn)
                    if not refine:
                        tr = jnp.maximum(t, 0.0)
                        ls = ls + jnp.where(pos, dsq, tr * tr)
                return ls

            return lax.fori_loop(0, VECS // UNROLL, vec_body, ls)

        issue(0, 0)
        issue(1, 1)

        def group_body(g, ls):
            for b in range(2):
                drain(b)
                ls = consume(b, ls)
                issue(g * 2 + b + 2, b)
            return ls

        ls = lax.fori_loop(0, NGRP - 1, group_body,
                           jnp.zeros((L,), jnp.float32))
        for b in range(2):
            drain(b)
            ls = consume(b, ls)

        pltpu.sync_copy(hist, hist_out.at[wid])
        if not refine:
            lstage[...] = ls
            pltpu.sync_copy(lstage, loss_out.at[wid])

    return body


_pass1 = pl.kernel(
    _make_pass(False),
    out_type=(
        jax.ShapeDtypeStruct((NW, 4 * HSZ), jnp.float32),
        jax.ShapeDtypeStruct((NW, L), jnp.float32),
    ),
    mesh=_mesh,
    compiler_params=_params,
    scratch_types=(
        pltpu.VMEM((CHUNK,), jnp.float32),
        pltpu.VMEM((CHUNK,), jnp.float32),
        pltpu.VMEM((CHUNK,), jnp.int32),
        pltpu.VMEM((CHUNK,), jnp.int32),
        pltpu.VMEM((4 * HSZ,), jnp.float32),
        pltpu.VMEM((L,), jnp.float32),
        pltpu.SemaphoreType.DMA,
        pltpu.SemaphoreType.DMA,
        pltpu.SemaphoreType.DMA,
        pltpu.SemaphoreType.DMA,
    ),
)

_pass2 = pl.kernel(
    _make_pass(True),
    out_type=jax.ShapeDtypeStruct((NW, 4 * HSZ), jnp.float32),
    mesh=_mesh,
    compiler_params=_params,
    scratch_types=(
        pltpu.VMEM((CHUNK,), jnp.float32),
        pltpu.VMEM((CHUNK,), jnp.float32),
        pltpu.VMEM((CHUNK,), jnp.int32),
        pltpu.VMEM((CHUNK,), jnp.int32),
        pltpu.VMEM((4 * HSZ,), jnp.float32),
        pltpu.VMEM((2 * L,), jnp.int32),
        pltpu.SemaphoreType.DMA,
        pltpu.SemaphoreType.DMA,
        pltpu.SemaphoreType.DMA,
        pltpu.SemaphoreType.DMA,
    ),
)


def _select_desc(cnt, ssum, m):
    suf_c = jnp.cumsum(cnt[::-1])[::-1]
    ok = suf_c >= m
    b = NB - 1 - jnp.argmax(ok[::-1])
    above_c = suf_c[b] - cnt[b]
    suf_s = jnp.cumsum(ssum[::-1])[::-1]
    above_s = suf_s[b] - ssum[b]
    return above_s, b, m - above_c


def _select_asc(cnt, ssum, m):
    pre_c = jnp.cumsum(cnt)
    ok = pre_c >= m
    b = jnp.argmax(ok)
    below_c = pre_c[b] - cnt[b]
    pre_s = jnp.cumsum(ssum)
    below_s = pre_s[b] - ssum[b]
    return below_s, b, m - below_c


def _frac_part(cnt_b, sum_b, r):
    safe = jnp.maximum(cnt_b, 1.0)
    return jnp.where(cnt_b > 0.0, (r / safe) * sum_b, 0.0)


def kernel(data, labels):
    d = data.reshape(-1)
    lab = labels.reshape(-1)

    hist1, loss_p = _pass1(d, lab)
    h1 = hist1.reshape(NW, 4, NB, L).sum(axis=(0, 3))
    pc, ps, nc, ns = h1[0], h1[1], h1[2], h1[3]
    n_pos = jnp.sum(pc)
    n_neg = jnp.sum(nc)
    m = jnp.minimum(n_pos, n_neg)

    ps_above, bp, need_p = _select_desc(pc, ps, m)
    ns_below, bn, need_n = _select_asc(nc, ns, m)

    thr = jnp.concatenate(
        [jnp.full((L,), bp, jnp.int32), jnp.full((L,), bn, jnp.int32)]
    )
    hist2 = _pass2(d, lab, thr)
    h2 = hist2.reshape(NW, 4, NB, L).sum(axis=(0, 3))
    pc2, ps2, nc2, ns2 = h2[0], h2[1], h2[2], h2[3]

    ps2_above, sp, r_p = _select_desc(pc2, ps2, need_p)
    ns2_below, sn, r_n = _select_asc(nc2, ns2, need_n)

    sum_pos = ps_above + ps2_above + _frac_part(pc2[sp], ps2[sp], r_p)
    sum_neg = ns_below + ns2_below + _frac_part(nc2[sn], ns2[sn], r_n)

    mined = (sum_pos + sum_neg) / (2.0 * m)
    loss_all = jnp.sum(loss_p) / jnp.float32(N_TOTAL)
    return jnp.where(n_neg == 0, loss_all, mined)


# speedup vs baseline: 71.0279x; 1.0755x over previous
import jax
import jax.numpy as jnp
from jax import lax
from jax.experimental import pallas as pl
from jax.experimental.pallas import tpu as pltpu
from jax.experimental.pallas import tpu_sc as plsc

MARGIN = 1.0
N_TOTAL = 16 * 3 * 512 * 512
NW = 32
L = 16
PER_W = N_TOTAL // NW
CHUNK = 8192
VECS = CHUNK // L
NCHUNK = PER_W // CHUNK
NGRP = NCHUNK // 2
UNROLL = 4
NB = 1024
HSZ = NB * L
B1_SHIFT = 22
B2_SHIFT = 12

_mesh = plsc.VectorSubcoreMesh(core_axis_name="c", subcore_axis_name="s")
_params = pltpu.CompilerParams(needs_layout_passes=False)


def _keys_bins(d, want2):
    bits = plsc.bitcast(d, jnp.int32)
    key = bits ^ ((bits >> 31) | jnp.int32(-(2**31)))
    keyu = plsc.bitcast(key, jnp.uint32)
    bin1 = (keyu >> jnp.uint32(B1_SHIFT)).astype(jnp.int32)
    bin2 = None
    if want2:
        bin2 = ((keyu >> jnp.uint32(B2_SHIFT)) & jnp.uint32(NB - 1)).astype(jnp.int32)
    return bin1, bin2


def _zero_hist(hist):
    zero = jnp.zeros((L,), jnp.float32)

    def zbody(i, carry):
        for u in range(8):
            hist[pl.ds((i * 8 + u) * L, L)] = zero
        return carry

    lax.fori_loop(0, 4 * HSZ // (8 * L), zbody, 0)


def _make_pass(refine):
    def body(data_hbm, labels_hbm, *rest):
        if refine:
            (thr_hbm, hist_out,
             d0, d1, l0, l1, hist, tbuf,
             sd0, sd1, sl0, sl1) = rest
        else:
            (hist_out,
             d0, d1, l0, l1, hist,
             sd0, sd1, sl0, sl1) = rest
        dbufs, lbufs = (d0, d1), (l0, l1)
        dsems, lsems = (sd0, sd1), (sl0, sl1)

        wid = lax.axis_index("s") * 2 + lax.axis_index("c")
        base = wid * PER_W
        _zero_hist(hist)
        if refine:
            pltpu.sync_copy(thr_hbm, tbuf)
            thrp = tbuf[pl.ds(0, L)]
            thrn = tbuf[pl.ds(L, L)]
        lane = lax.iota(jnp.int32, L)
        ones = jnp.ones((L,), jnp.float32)
        off_neg = jnp.full((L,), 2 * HSZ, jnp.int32)

        def issue(c, b):
            start = base + c * CHUNK
            pltpu.async_copy(data_hbm.at[pl.ds(start, CHUNK)], dbufs[b], dsems[b])
            pltpu.async_copy(labels_hbm.at[pl.ds(start, CHUNK)], lbufs[b], lsems[b])

        def drain(b):
            pltpu.make_async_copy(
                data_hbm.at[pl.ds(0, CHUNK)], dbufs[b], dsems[b]).wait()
            pltpu.make_async_copy(
                labels_hbm.at[pl.ds(0, CHUNK)], lbufs[b], lsems[b]).wait()

        def consume(b):
            dbuf, lbuf = dbufs[b], lbufs[b]

            def vec_body(v, carry):
                for u in range(UNROLL):
                    off = (v * UNROLL + u) * L
                    d = dbuf[pl.ds(off, L)]
                    lab = lbuf[pl.ds(off, L)]
                    pos = lab != 0
                    hneg = jnp.logical_and(lab == 0, d <= MARGIN)
                    bin1, bin2 = _keys_bins(d, refine)
                    dsq = d * d
                    t = MARGIN - d
                    tsq = t * t
                    if refine:
                        mp = jnp.logical_and(pos, bin1 == thrp)
                        mn = jnp.logical_and(hneg, bin1 == thrn)
                        idx = bin2 * L + lane
                    else:
                        mp, mn = pos, hneg
                        idx = bin1 * L + lane
                    any_m = jnp.logical_or(mp, mn)
                    idx_c = idx + jnp.where(mp, 0, off_neg)
                    val = jnp.where(mp, dsq, tsq)
                    plsc.addupdate_scatter(hist, (idx_c,), ones, mask=any_m)
                    plsc.addupdate_scatter(hist, (idx_c + HSZ,), val, mask=any_m)
                return carry

            lax.fori_loop(0, VECS // UNROLL, vec_body, 0)

        issue(0, 0)
        issue(1, 1)

        def group_body(g, carry):
            for b in range(2):
                drain(b)
                consume(b)
                issue(g * 2 + b + 2, b)
            return carry

        lax.fori_loop(0, NGRP - 1, group_body, 0)
        for b in range(2):
            drain(b)
            consume(b)

        pltpu.sync_copy(hist, hist_out.at[wid])

    return body


_scratch_common = (
    pltpu.VMEM((CHUNK,), jnp.float32),
    pltpu.VMEM((CHUNK,), jnp.float32),
    pltpu.VMEM((CHUNK,), jnp.int32),
    pltpu.VMEM((CHUNK,), jnp.int32),
    pltpu.VMEM((4 * HSZ,), jnp.float32),
)
_sems = (
    pltpu.SemaphoreType.DMA,
    pltpu.SemaphoreType.DMA,
    pltpu.SemaphoreType.DMA,
    pltpu.SemaphoreType.DMA,
)

_pass1 = pl.kernel(
    _make_pass(False),
    out_type=jax.ShapeDtypeStruct((NW, 4 * HSZ), jnp.float32),
    mesh=_mesh,
    compiler_params=_params,
    scratch_types=_scratch_common + _sems,
)

_pass2 = pl.kernel(
    _make_pass(True),
    out_type=jax.ShapeDtypeStruct((NW, 4 * HSZ), jnp.float32),
    mesh=_mesh,
    compiler_params=_params,
    scratch_types=_scratch_common + (pltpu.VMEM((2 * L,), jnp.int32),) + _sems,
)


def _select_desc(cnt, ssum, m):
    suf_c = jnp.cumsum(cnt[::-1])[::-1]
    ok = suf_c >= m
    b = NB - 1 - jnp.argmax(ok[::-1])
    above_c = suf_c[b] - cnt[b]
    suf_s = jnp.cumsum(ssum[::-1])[::-1]
    above_s = suf_s[b] - ssum[b]
    return above_s, b, m - above_c


def _select_asc(cnt, ssum, m):
    pre_c = jnp.cumsum(cnt)
    ok = pre_c >= m
    b = jnp.argmax(ok)
    below_c = pre_c[b] - cnt[b]
    pre_s = jnp.cumsum(ssum)
    below_s = pre_s[b] - ssum[b]
    return below_s, b, m - below_c


def _frac_part(cnt_b, sum_b, r):
    safe = jnp.maximum(cnt_b, 1.0)
    return jnp.where(cnt_b > 0.0, (r / safe) * sum_b, 0.0)


def kernel(data, labels):
    d = data.reshape(-1)
    lab = labels.reshape(-1)

    hist1 = _pass1(d, lab)
    h1 = hist1.reshape(NW, 4, NB, L).sum(axis=(0, 3))
    pc, ps, nc, ns = h1[0], h1[1], h1[2], h1[3]
    n_pos = jnp.sum(pc)
    n_neg = jnp.sum(nc)
    m = jnp.minimum(n_pos, n_neg)

    ps_above, bp, need_p = _select_desc(pc, ps, m)
    ns_below, bn, need_n = _select_asc(nc, ns, m)

    thr = jnp.concatenate(
        [jnp.full((L,), bp, jnp.int32), jnp.full((L,), bn, jnp.int32)]
    )
    hist2 = _pass2(d, lab, thr)
    h2 = hist2.reshape(NW, 4, NB, L).sum(axis=(0, 3))
    pc2, ps2, nc2, ns2 = h2[0], h2[1], h2[2], h2[3]

    ps2_above, sp, r_p = _select_desc(pc2, ps2, need_p)
    ns2_below, sn, r_n = _select_asc(nc2, ns2, need_n)

    sum_pos = ps_above + ps2_above + _frac_part(pc2[sp], ps2[sp], r_p)
    sum_neg = ns_below + ns2_below + _frac_part(nc2[sn], ns2[sn], r_n)

    mined = (sum_pos + sum_neg) / (2.0 * m)
    loss_all = (jnp.sum(ps) + jnp.sum(ns)) / jnp.float32(N_TOTAL)
    return jnp.where(n_neg == 0, loss_all, mined)


# speedup vs baseline: 71.6561x; 1.0088x over previous
import jax
import jax.numpy as jnp
from jax import lax
from jax.experimental import pallas as pl
from jax.experimental.pallas import tpu as pltpu
from jax.experimental.pallas import tpu_sc as plsc

MARGIN = 1.0
N_TOTAL = 16 * 3 * 512 * 512
NW = 32
L = 16
PER_W = N_TOTAL // NW
CHUNK = 8192
VECS = CHUNK // L
NCHUNK = PER_W // CHUNK
NGRP = NCHUNK // 2
UNROLL = 8
NB = 1024
HSZ = NB * L
B1_SHIFT = 22
B2_SHIFT = 12

_mesh = plsc.VectorSubcoreMesh(core_axis_name="c", subcore_axis_name="s")
_params = pltpu.CompilerParams(needs_layout_passes=False)


def _keys_bins(d, want2):
    bits = plsc.bitcast(d, jnp.int32)
    key = bits ^ ((bits >> 31) | jnp.int32(-(2**31)))
    keyu = plsc.bitcast(key, jnp.uint32)
    bin1 = (keyu >> jnp.uint32(B1_SHIFT)).astype(jnp.int32)
    bin2 = None
    if want2:
        bin2 = ((keyu >> jnp.uint32(B2_SHIFT)) & jnp.uint32(NB - 1)).astype(jnp.int32)
    return bin1, bin2


def _zero_hist(hist):
    zero = jnp.zeros((L,), jnp.float32)

    def zbody(i, carry):
        for u in range(8):
            hist[pl.ds((i * 8 + u) * L, L)] = zero
        return carry

    lax.fori_loop(0, 4 * HSZ // (8 * L), zbody, 0)


def _make_pass(refine):
    def body(data_hbm, labels_hbm, *rest):
        if refine:
            (thr_hbm, hist_out,
             d0, d1, l0, l1, hist, tbuf,
             sd0, sd1, sl0, sl1) = rest
        else:
            (hist_out,
             d0, d1, l0, l1, hist,
             sd0, sd1, sl0, sl1) = rest
        dbufs, lbufs = (d0, d1), (l0, l1)
        dsems, lsems = (sd0, sd1), (sl0, sl1)

        wid = lax.axis_index("s") * 2 + lax.axis_index("c")
        base = wid * PER_W
        _zero_hist(hist)
        if refine:
            pltpu.sync_copy(thr_hbm, tbuf)
            thrp = tbuf[pl.ds(0, L)]
            thrn = tbuf[pl.ds(L, L)]
        lane = lax.iota(jnp.int32, L)
        ones = jnp.ones((L,), jnp.float32)
        off_neg = jnp.full((L,), 2 * HSZ, jnp.int32)

        def issue(c, b):
            start = base + c * CHUNK
            pltpu.async_copy(data_hbm.at[pl.ds(start, CHUNK)], dbufs[b], dsems[b])
            pltpu.async_copy(labels_hbm.at[pl.ds(start, CHUNK)], lbufs[b], lsems[b])

        def drain(b):
            pltpu.make_async_copy(
                data_hbm.at[pl.ds(0, CHUNK)], dbufs[b], dsems[b]).wait()
            pltpu.make_async_copy(
                labels_hbm.at[pl.ds(0, CHUNK)], lbufs[b], lsems[b]).wait()

        def consume(b):
            dbuf, lbuf = dbufs[b], lbufs[b]

            def vec_body(v, carry):
                for u in range(UNROLL):
                    off = (v * UNROLL + u) * L
                    d = dbuf[pl.ds(off, L)]
                    lab = lbuf[pl.ds(off, L)]
                    pos = lab != 0
                    hneg = jnp.logical_and(lab == 0, d <= MARGIN)
                    bin1, bin2 = _keys_bins(d, refine)
                    dsq = d * d
                    t = MARGIN - d
                    tsq = t * t
                    if refine:
                        mp = jnp.logical_and(pos, bin1 == thrp)
                        mn = jnp.logical_and(hneg, bin1 == thrn)
                        idx = bin2 * L + lane
                    else:
                        mp, mn = pos, hneg
                        idx = bin1 * L + lane
                    any_m = jnp.logical_or(mp, mn)
                    idx_c = idx + jnp.where(mp, 0, off_neg)
                    val = jnp.where(mp, dsq, tsq)
                    plsc.addupdate_scatter(hist, (idx_c,), ones, mask=any_m)
                    plsc.addupdate_scatter(hist, (idx_c + HSZ,), val, mask=any_m)
                return carry

            lax.fori_loop(0, VECS // UNROLL, vec_body, 0)

        issue(0, 0)
        issue(1, 1)

        def group_body(g, carry):
            for b in range(2):
                drain(b)
                consume(b)
                issue(g * 2 + b + 2, b)
            return carry

        lax.fori_loop(0, NGRP - 1, group_body, 0)
        for b in range(2):
            drain(b)
            consume(b)

        pltpu.sync_copy(hist, hist_out.at[wid])

    return body


_scratch_common = (
    pltpu.VMEM((CHUNK,), jnp.float32),
    pltpu.VMEM((CHUNK,), jnp.float32),
    pltpu.VMEM((CHUNK,), jnp.int32),
    pltpu.VMEM((CHUNK,), jnp.int32),
    pltpu.VMEM((4 * HSZ,), jnp.float32),
)
_sems = (
    pltpu.SemaphoreType.DMA,
    pltpu.SemaphoreType.DMA,
    pltpu.SemaphoreType.DMA,
    pltpu.SemaphoreType.DMA,
)

_pass1 = pl.kernel(
    _make_pass(False),
    out_type=jax.ShapeDtypeStruct((NW, 4 * HSZ), jnp.float32),
    mesh=_mesh,
    compiler_params=_params,
    scratch_types=_scratch_common + _sems,
)

_pass2 = pl.kernel(
    _make_pass(True),
    out_type=jax.ShapeDtypeStruct((NW, 4 * HSZ), jnp.float32),
    mesh=_mesh,
    compiler_params=_params,
    scratch_types=_scratch_common + (pltpu.VMEM((2 * L,), jnp.int32),) + _sems,
)


def _select_desc(cnt, ssum, m):
    suf_c = jnp.cumsum(cnt[::-1])[::-1]
    ok = suf_c >= m
    b = NB - 1 - jnp.argmax(ok[::-1])
    above_c = suf_c[b] - cnt[b]
    suf_s = jnp.cumsum(ssum[::-1])[::-1]
    above_s = suf_s[b] - ssum[b]
    return above_s, b, m - above_c


def _select_asc(cnt, ssum, m):
    pre_c = jnp.cumsum(cnt)
    ok = pre_c >= m
    b = jnp.argmax(ok)
    below_c = pre_c[b] - cnt[b]
    pre_s = jnp.cumsum(ssum)
    below_s = pre_s[b] - ssum[b]
    return below_s, b, m - below_c


def _frac_part(cnt_b, sum_b, r):
    safe = jnp.maximum(cnt_b, 1.0)
    return jnp.where(cnt_b > 0.0, (r / safe) * sum_b, 0.0)


def kernel(data, labels):
    d = data.reshape(-1)
    lab = labels.reshape(-1)

    hist1 = _pass1(d, lab)
    h1 = hist1.reshape(NW, 4, NB, L).sum(axis=(0, 3))
    pc, ps, nc, ns = h1[0], h1[1], h1[2], h1[3]
    n_pos = jnp.sum(pc)
    n_neg = jnp.sum(nc)
    m = jnp.minimum(n_pos, n_neg)

    ps_above, bp, need_p = _select_desc(pc, ps, m)
    ns_below, bn, need_n = _select_asc(nc, ns, m)

    thr = jnp.concatenate(
        [jnp.full((L,), bp, jnp.int32), jnp.full((L,), bn, jnp.int32)]
    )
    hist2 = _pass2(d, lab, thr)
    h2 = hist2.reshape(NW, 4, NB, L).sum(axis=(0, 3))
    pc2, ps2, nc2, ns2 = h2[0], h2[1], h2[2], h2[3]

    ps2_above, sp, r_p = _select_desc(pc2, ps2, need_p)
    ns2_below, sn, r_n = _select_asc(nc2, ns2, need_n)

    sum_pos = ps_above + ps2_above + _frac_part(pc2[sp], ps2[sp], r_p)
    sum_neg = ns_below + ns2_below + _frac_part(nc2[sn], ns2[sn], r_n)

    mined = (sum_pos + sum_neg) / (2.0 * m)
    loss_all = (jnp.sum(ps) + jnp.sum(ns)) / jnp.float32(N_TOTAL)
    return jnp.where(n_neg == 0, loss_all, mined)


# speedup vs baseline: 123.5699x; 1.7245x over previous
import jax
import jax.numpy as jnp
from jax import lax
from jax.experimental import pallas as pl
from jax.experimental.pallas import tpu as pltpu
from jax.experimental.pallas import tpu_sc as plsc

MARGIN = 1.0
N_TOTAL = 16 * 3 * 512 * 512
NW = 32
L = 16
PER_W = N_TOTAL // NW
CHUNK = 8192
VECS = CHUNK // L
NCHUNK = PER_W // CHUNK
NGRP = NCHUNK // 2
UNROLL = 8
NB = 1024
HSZ = NB * L
B1_SHIFT = 22
B2_SHIFT = 12

_mesh = plsc.VectorSubcoreMesh(core_axis_name="c", subcore_axis_name="s")
_params = pltpu.CompilerParams(needs_layout_passes=False)


def _keys_bins(d, want2):
    bits = plsc.bitcast(d, jnp.int32)
    key = bits ^ ((bits >> 31) | jnp.int32(-(2**31)))
    keyu = plsc.bitcast(key, jnp.uint32)
    bin1 = (keyu >> jnp.uint32(B1_SHIFT)).astype(jnp.int32)
    bin2 = None
    if want2:
        bin2 = ((keyu >> jnp.uint32(B2_SHIFT)) & jnp.uint32(NB - 1)).astype(jnp.int32)
    return bin1, bin2


def _zero_hist(hist):
    zero = jnp.zeros((L,), jnp.float32)

    def zbody(i, carry):
        for u in range(8):
            hist[pl.ds((i * 8 + u) * L, L)] = zero
        return carry

    lax.fori_loop(0, 4 * HSZ // (8 * L), zbody, 0)


def _make_pass(refine):
    def body(data_hbm, labels_hbm, *rest):
        if refine:
            (thr_hbm, hist_out,
             d0, d1, l0, l1, hist, tbuf,
             sd0, sd1, sl0, sl1) = rest
        else:
            (hist_out,
             d0, d1, l0, l1, hist,
             sd0, sd1, sl0, sl1) = rest
        dbufs, lbufs = (d0, d1), (l0, l1)
        dsems, lsems = (sd0, sd1), (sl0, sl1)

        wid = lax.axis_index("s") * 2 + lax.axis_index("c")
        base = wid * PER_W
        _zero_hist(hist)
        if refine:
            pltpu.sync_copy(thr_hbm, tbuf)
            thrp = tbuf[pl.ds(0, L)]
            thrn = tbuf[pl.ds(L, L)]
        lane = lax.iota(jnp.int32, L)
        ones = jnp.ones((L,), jnp.float32)
        off_neg = jnp.full((L,), 2 * HSZ, jnp.int32)

        def issue(c, b):
            start = base + c * CHUNK
            pltpu.async_copy(data_hbm.at[pl.ds(start, CHUNK)], dbufs[b], dsems[b])
            pltpu.async_copy(labels_hbm.at[pl.ds(start, CHUNK)], lbufs[b], lsems[b])

        def drain(b):
            pltpu.make_async_copy(
                data_hbm.at[pl.ds(0, CHUNK)], dbufs[b], dsems[b]).wait()
            pltpu.make_async_copy(
                labels_hbm.at[pl.ds(0, CHUNK)], lbufs[b], lsems[b]).wait()

        def consume(b):
            dbuf, lbuf = dbufs[b], lbufs[b]

            @plsc.parallel_loop(0, VECS, step=1, unroll=UNROLL)
            def _(v):
                off = v * L
                d = dbuf[pl.ds(off, L)]
                lab = lbuf[pl.ds(off, L)]
                pos = lab != 0
                hneg = jnp.logical_and(lab == 0, d <= MARGIN)
                bin1, bin2 = _keys_bins(d, refine)
                dsq = d * d
                t = MARGIN - d
                tsq = t * t
                if refine:
                    mp = jnp.logical_and(pos, bin1 == thrp)
                    mn = jnp.logical_and(hneg, bin1 == thrn)
                    idx = bin2 * L + lane
                else:
                    mp, mn = pos, hneg
                    idx = bin1 * L + lane
                any_m = jnp.logical_or(mp, mn)
                idx_c = idx + jnp.where(mp, 0, off_neg)
                val = jnp.where(mp, dsq, tsq)
                plsc.addupdate_scatter(hist, (idx_c,), ones, mask=any_m)
                plsc.addupdate_scatter(hist, (idx_c + HSZ,), val, mask=any_m)

        issue(0, 0)
        issue(1, 1)

        def group_body(g, carry):
            for b in range(2):
                drain(b)
                consume(b)
                issue(g * 2 + b + 2, b)
            return carry

        lax.fori_loop(0, NGRP - 1, group_body, 0)
        for b in range(2):
            drain(b)
            consume(b)

        pltpu.sync_copy(hist, hist_out.at[wid])

    return body


_scratch_common = (
    pltpu.VMEM((CHUNK,), jnp.float32),
    pltpu.VMEM((CHUNK,), jnp.float32),
    pltpu.VMEM((CHUNK,), jnp.int32),
    pltpu.VMEM((CHUNK,), jnp.int32),
    pltpu.VMEM((4 * HSZ,), jnp.float32),
)
_sems = (
    pltpu.SemaphoreType.DMA,
    pltpu.SemaphoreType.DMA,
    pltpu.SemaphoreType.DMA,
    pltpu.SemaphoreType.DMA,
)

_pass1 = pl.kernel(
    _make_pass(False),
    out_type=jax.ShapeDtypeStruct((NW, 4 * HSZ), jnp.float32),
    mesh=_mesh,
    compiler_params=_params,
    scratch_types=_scratch_common + _sems,
)

_pass2 = pl.kernel(
    _make_pass(True),
    out_type=jax.ShapeDtypeStruct((NW, 4 * HSZ), jnp.float32),
    mesh=_mesh,
    compiler_params=_params,
    scratch_types=_scratch_common + (pltpu.VMEM((2 * L,), jnp.int32),) + _sems,
)


def _select_desc(cnt, ssum, m):
    suf_c = jnp.cumsum(cnt[::-1])[::-1]
    ok = suf_c >= m
    b = NB - 1 - jnp.argmax(ok[::-1])
    above_c = suf_c[b] - cnt[b]
    suf_s = jnp.cumsum(ssum[::-1])[::-1]
    above_s = suf_s[b] - ssum[b]
    return above_s, b, m - above_c


def _select_asc(cnt, ssum, m):
    pre_c = jnp.cumsum(cnt)
    ok = pre_c >= m
    b = jnp.argmax(ok)
    below_c = pre_c[b] - cnt[b]
    pre_s = jnp.cumsum(ssum)
    below_s = pre_s[b] - ssum[b]
    return below_s, b, m - below_c


def _frac_part(cnt_b, sum_b, r):
    safe = jnp.maximum(cnt_b, 1.0)
    return jnp.where(cnt_b > 0.0, (r / safe) * sum_b, 0.0)


def kernel(data, labels):
    d = data.reshape(-1)
    lab = labels.reshape(-1)

    hist1 = _pass1(d, lab)
    h1 = hist1.reshape(NW, 4, NB, L).sum(axis=(0, 3))
    pc, ps, nc, ns = h1[0], h1[1], h1[2], h1[3]
    n_pos = jnp.sum(pc)
    n_neg = jnp.sum(nc)
    m = jnp.minimum(n_pos, n_neg)

    ps_above, bp, need_p = _select_desc(pc, ps, m)
    ns_below, bn, need_n = _select_asc(nc, ns, m)

    thr = jnp.concatenate(
        [jnp.full((L,), bp, jnp.int32), jnp.full((L,), bn, jnp.int32)]
    )
    hist2 = _pass2(d, lab, thr)
    h2 = hist2.reshape(NW, 4, NB, L).sum(axis=(0, 3))
    pc2, ps2, nc2, ns2 = h2[0], h2[1], h2[2], h2[3]

    ps2_above, sp, r_p = _select_desc(pc2, ps2, need_p)
    ns2_below, sn, r_n = _select_asc(nc2, ns2, need_n)

    sum_pos = ps_above + ps2_above + _frac_part(pc2[sp], ps2[sp], r_p)
    sum_neg = ns_below + ns2_below + _frac_part(nc2[sn], ns2[sn], r_n)

    mined = (sum_pos + sum_neg) / (2.0 * m)
    loss_all = (jnp.sum(ps) + jnp.sum(ns)) / jnp.float32(N_TOTAL)
    return jnp.where(n_neg == 0, loss_all, mined)


# speedup vs baseline: 133.4783x; 1.0802x over previous
import jax
import jax.numpy as jnp
from jax import lax
from jax.experimental import pallas as pl
from jax.experimental.pallas import tpu as pltpu
from jax.experimental.pallas import tpu_sc as plsc

MARGIN = 1.0
N_TOTAL = 16 * 3 * 512 * 512
NW = 32
L = 16
PER_W = N_TOTAL // NW
CHUNK = 8192
VECS = CHUNK // L
NCHUNK = PER_W // CHUNK
NGRP = NCHUNK // 2
UNROLL = 8
NB = 1024
HSZ = NB * L
B1_SHIFT = 22
B2_SHIFT = 12

_mesh = plsc.VectorSubcoreMesh(core_axis_name="c", subcore_axis_name="s")
_params = pltpu.CompilerParams(needs_layout_passes=False)


def _keys_bins(d, want2):
    bits = plsc.bitcast(d, jnp.int32)
    key = bits ^ ((bits >> 31) | jnp.int32(-(2**31)))
    keyu = plsc.bitcast(key, jnp.uint32)
    bin1 = (keyu >> jnp.uint32(B1_SHIFT)).astype(jnp.int32)
    bin2 = None
    if want2:
        bin2 = ((keyu >> jnp.uint32(B2_SHIFT)) & jnp.uint32(NB - 1)).astype(jnp.int32)
    return bin1, bin2


def _zero_hist(hist):
    zero = jnp.zeros((L,), jnp.float32)

    def zbody(i, carry):
        for u in range(8):
            hist[pl.ds((i * 8 + u) * L, L)] = zero
        return carry

    lax.fori_loop(0, 4 * HSZ // (8 * L), zbody, 0)


def _make_pass(refine):
    def body(data_hbm, labels_hbm, *rest):
        if refine:
            (thr_hbm, hist_out,
             d0, d1, l0, l1, hist, tbuf,
             sd0, sd1, sl0, sl1) = rest
        else:
            (hist_out,
             d0, d1, l0, l1, hist,
             sd0, sd1, sl0, sl1) = rest
        dbufs, lbufs = (d0, d1), (l0, l1)
        dsems, lsems = (sd0, sd1), (sl0, sl1)

        wid = lax.axis_index("s") * 2 + lax.axis_index("c")
        base = wid * PER_W
        _zero_hist(hist)
        if refine:
            pltpu.sync_copy(thr_hbm, tbuf)
            thrp = tbuf[pl.ds(0, L)]
            thrn = tbuf[pl.ds(L, L)]
        lane = lax.iota(jnp.int32, L)
        ones = jnp.ones((L,), jnp.float32)
        off_neg = jnp.full((L,), 2 * HSZ, jnp.int32)

        def issue(c, b):
            start = base + c * CHUNK
            pltpu.async_copy(data_hbm.at[pl.ds(start, CHUNK)], dbufs[b], dsems[b])
            pltpu.async_copy(labels_hbm.at[pl.ds(start, CHUNK)], lbufs[b], lsems[b])

        def drain(b):
            pltpu.make_async_copy(
                data_hbm.at[pl.ds(0, CHUNK)], dbufs[b], dsems[b]).wait()
            pltpu.make_async_copy(
                labels_hbm.at[pl.ds(0, CHUNK)], lbufs[b], lsems[b]).wait()

        def consume(b):
            dbuf, lbuf = dbufs[b], lbufs[b]

            @plsc.parallel_loop(0, VECS, step=1, unroll=UNROLL)
            def _(v):
                off = v * L
                d = dbuf[pl.ds(off, L)]
                lab = lbuf[pl.ds(off, L)]
                pos = lab != 0
                bin1, bin2 = _keys_bins(d, refine)
                t = MARGIN - d
                if refine:
                    hneg = jnp.logical_and(lab == 0, d <= MARGIN)
                    mp = jnp.logical_and(pos, bin1 == thrp)
                    mn = jnp.logical_and(hneg, bin1 == thrn)
                    any_m = jnp.logical_or(mp, mn)
                    idx = bin2 * L + lane
                else:
                    mp = pos
                    any_m = jnp.logical_or(pos, d <= MARGIN)
                    idx = bin1 * L + lane
                base_v = jnp.where(mp, d, t)
                val = base_v * base_v
                idx_c = idx + jnp.where(mp, 0, off_neg)
                plsc.addupdate_scatter(hist, (idx_c,), ones, mask=any_m)
                plsc.addupdate_scatter(hist, (idx_c + HSZ,), val, mask=any_m)

        issue(0, 0)
        issue(1, 1)

        def group_body(g, carry):
            for b in range(2):
                drain(b)
                consume(b)
                issue(g * 2 + b + 2, b)
            return carry

        lax.fori_loop(0, NGRP - 1, group_body, 0)
        for b in range(2):
            drain(b)
            consume(b)

        pltpu.sync_copy(hist, hist_out.at[wid])

    return body


_scratch_common = (
    pltpu.VMEM((CHUNK,), jnp.float32),
    pltpu.VMEM((CHUNK,), jnp.float32),
    pltpu.VMEM((CHUNK,), jnp.int32),
    pltpu.VMEM((CHUNK,), jnp.int32),
    pltpu.VMEM((4 * HSZ,), jnp.float32),
)
_sems = (
    pltpu.SemaphoreType.DMA,
    pltpu.SemaphoreType.DMA,
    pltpu.SemaphoreType.DMA,
    pltpu.SemaphoreType.DMA,
)

_pass1 = pl.kernel(
    _make_pass(False),
    out_type=jax.ShapeDtypeStruct((NW, 4 * HSZ), jnp.float32),
    mesh=_mesh,
    compiler_params=_params,
    scratch_types=_scratch_common + _sems,
)

_pass2 = pl.kernel(
    _make_pass(True),
    out_type=jax.ShapeDtypeStruct((NW, 4 * HSZ), jnp.float32),
    mesh=_mesh,
    compiler_params=_params,
    scratch_types=_scratch_common + (pltpu.VMEM((2 * L,), jnp.int32),) + _sems,
)


def _select_desc(cnt, ssum, m):
    suf_c = jnp.cumsum(cnt[::-1])[::-1]
    ok = suf_c >= m
    b = NB - 1 - jnp.argmax(ok[::-1])
    above_c = suf_c[b] - cnt[b]
    suf_s = jnp.cumsum(ssum[::-1])[::-1]
    above_s = suf_s[b] - ssum[b]
    return above_s, b, m - above_c


def _select_asc(cnt, ssum, m):
    pre_c = jnp.cumsum(cnt)
    ok = pre_c >= m
    b = jnp.argmax(ok)
    below_c = pre_c[b] - cnt[b]
    pre_s = jnp.cumsum(ssum)
    below_s = pre_s[b] - ssum[b]
    return below_s, b, m - below_c


def _frac_part(cnt_b, sum_b, r):
    safe = jnp.maximum(cnt_b, 1.0)
    return jnp.where(cnt_b > 0.0, (r / safe) * sum_b, 0.0)


def kernel(data, labels):
    d = data.reshape(-1)
    lab = labels.reshape(-1)

    hist1 = _pass1(d, lab)
    h1 = hist1.reshape(NW, 4, NB, L).sum(axis=(0, 3))
    pc, ps, nc, ns = h1[0], h1[1], h1[2], h1[3]
    n_pos = jnp.sum(pc)
    n_neg = jnp.sum(nc)
    m = jnp.minimum(n_pos, n_neg)

    ps_above, bp, need_p = _select_desc(pc, ps, m)
    ns_below, bn, need_n = _select_asc(nc, ns, m)

    thr = jnp.concatenate(
        [jnp.full((L,), bp, jnp.int32), jnp.full((L,), bn, jnp.int32)]
    )
    hist2 = _pass2(d, lab, thr)
    h2 = hist2.reshape(NW, 4, NB, L).sum(axis=(0, 3))
    pc2, ps2, nc2, ns2 = h2[0], h2[1], h2[2], h2[3]

    ps2_above, sp, r_p = _select_desc(pc2, ps2, need_p)
    ns2_below, sn, r_n = _select_asc(nc2, ns2, need_n)

    sum_pos = ps_above + ps2_above + _frac_part(pc2[sp], ps2[sp], r_p)
    sum_neg = ns_below + ns2_below + _frac_part(nc2[sn], ns2[sn], r_n)

    mined = (sum_pos + sum_neg) / (2.0 * m)
    loss_all = (jnp.sum(ps) + jnp.sum(ns)) / jnp.float32(N_TOTAL)
    return jnp.where(n_neg == 0, loss_all, mined)


# speedup vs baseline: 161.2763x; 1.2083x over previous
import jax
import jax.numpy as jnp
from jax import lax
from jax.experimental import pallas as pl
from jax.experimental.pallas import tpu as pltpu
from jax.experimental.pallas import tpu_sc as plsc

MARGIN = 1.0
N_TOTAL = 16 * 3 * 512 * 512
NW = 32
L = 16
PER_W = N_TOTAL // NW
CHUNK = 8192
VECS = CHUNK // L
NCHUNK = PER_W // CHUNK
NGRP = NCHUNK // 2
UNROLL = 8
NB = 1024
HSZ = NB * L
B1_SHIFT = 22
B2_SHIFT = 12

_mesh = plsc.VectorSubcoreMesh(core_axis_name="c", subcore_axis_name="s")
_params = pltpu.CompilerParams(needs_layout_passes=False)


def _keys_bins(d, want2):
    bits = plsc.bitcast(d, jnp.int32)
    key = bits ^ ((bits >> 31) | jnp.int32(-(2**31)))
    keyu = plsc.bitcast(key, jnp.uint32)
    bin1 = (keyu >> jnp.uint32(B1_SHIFT)).astype(jnp.int32)
    bin2 = None
    if want2:
        bin2 = ((keyu >> jnp.uint32(B2_SHIFT)) & jnp.uint32(NB - 1)).astype(jnp.int32)
    return bin1, bin2


def _zero_hist(hist):
    zero = jnp.zeros((L,), jnp.float32)

    def zbody(i, carry):
        for u in range(8):
            hist[pl.ds((i * 8 + u) * L, L)] = zero
        return carry

    lax.fori_loop(0, 4 * HSZ // (8 * L), zbody, 0)


def _make_pass(refine):
    def body(data_hbm, labels_hbm, *rest):
        if refine:
            (thr_hbm, hist_out,
             d0, d1, l0, l1, hist, tbuf,
             sd0, sd1, sl0, sl1) = rest
        else:
            (hist_out,
             d0, d1, l0, l1, hist,
             sd0, sd1, sl0, sl1) = rest
        dbufs, lbufs = (d0, d1), (l0, l1)
        dsems, lsems = (sd0, sd1), (sl0, sl1)

        wid = lax.axis_index("s") * 2 + lax.axis_index("c")
        base = wid * PER_W
        _zero_hist(hist)
        if refine:
            pltpu.sync_copy(thr_hbm, tbuf)
            thrp = tbuf[pl.ds(0, L)]
            thrn = tbuf[pl.ds(L, L)]
        lane = lax.iota(jnp.int32, L)
        ones = jnp.ones((L,), jnp.float32)
        off_neg = jnp.full((L,), 2 * HSZ, jnp.int32)

        def issue(c, b):
            start = base + c * CHUNK
            pltpu.async_copy(data_hbm.at[pl.ds(start, CHUNK)], dbufs[b], dsems[b])
            pltpu.async_copy(labels_hbm.at[pl.ds(start, CHUNK)], lbufs[b], lsems[b])

        def drain(b):
            pltpu.make_async_copy(
                data_hbm.at[pl.ds(0, CHUNK)], dbufs[b], dsems[b]).wait()
            pltpu.make_async_copy(
                labels_hbm.at[pl.ds(0, CHUNK)], lbufs[b], lsems[b]).wait()

        def consume(b):
            dbuf, lbuf = dbufs[b], lbufs[b]

            @plsc.parallel_loop(0, VECS, step=1, unroll=UNROLL)
            def _(v):
                off = v * L
                d = dbuf[pl.ds(off, L)]
                lab = lbuf[pl.ds(off, L)]
                pos = lab != 0
                bin1, bin2 = _keys_bins(d, refine)
                t = MARGIN - d
                if refine:
                    valid = jnp.logical_or(pos, d <= MARGIN)
                    thr_b = jnp.where(pos, thrp, thrn)
                    mp = pos
                    any_m = jnp.logical_and(valid, bin1 == thr_b)
                    idx = bin2 * L + lane
                else:
                    mp = pos
                    any_m = jnp.logical_or(pos, d <= MARGIN)
                    idx = bin1 * L + lane
                base_v = jnp.where(mp, d, t)
                val = base_v * base_v
                idx_c = idx + jnp.where(mp, 0, off_neg)
                plsc.addupdate_scatter(hist, (idx_c,), ones, mask=any_m)
                plsc.addupdate_scatter(hist, (idx_c + HSZ,), val, mask=any_m)

        issue(0, 0)
        issue(1, 1)

        def group_body(g, carry):
            for b in range(2):
                drain(b)
                consume(b)
                issue(g * 2 + b + 2, b)
            return carry

        lax.fori_loop(0, NGRP - 1, group_body, 0)
        for b in range(2):
            drain(b)
            consume(b)

        pltpu.sync_copy(hist, hist_out.at[wid])

    return body


_scratch_common = (
    pltpu.VMEM((CHUNK,), jnp.float32),
    pltpu.VMEM((CHUNK,), jnp.float32),
    pltpu.VMEM((CHUNK,), jnp.int32),
    pltpu.VMEM((CHUNK,), jnp.int32),
    pltpu.VMEM((4 * HSZ,), jnp.float32),
)
_sems = (
    pltpu.SemaphoreType.DMA,
    pltpu.SemaphoreType.DMA,
    pltpu.SemaphoreType.DMA,
    pltpu.SemaphoreType.DMA,
)

_pass1 = pl.kernel(
    _make_pass(False),
    out_type=jax.ShapeDtypeStruct((NW, 4 * HSZ), jnp.float32),
    mesh=_mesh,
    compiler_params=_params,
    scratch_types=_scratch_common + _sems,
)

_pass2 = pl.kernel(
    _make_pass(True),
    out_type=jax.ShapeDtypeStruct((NW, 4 * HSZ), jnp.float32),
    mesh=_mesh,
    compiler_params=_params,
    scratch_types=_scratch_common + (pltpu.VMEM((2 * L,), jnp.int32),) + _sems,
)


def _select_desc(cnt, ssum, m):
    suf_c = jnp.cumsum(cnt[::-1])[::-1]
    ok = suf_c >= m
    b = NB - 1 - jnp.argmax(ok[::-1])
    above_c = suf_c[b] - cnt[b]
    suf_s = jnp.cumsum(ssum[::-1])[::-1]
    above_s = suf_s[b] - ssum[b]
    return above_s, b, m - above_c


def _select_asc(cnt, ssum, m):
    pre_c = jnp.cumsum(cnt)
    ok = pre_c >= m
    b = jnp.argmax(ok)
    below_c = pre_c[b] - cnt[b]
    pre_s = jnp.cumsum(ssum)
    below_s = pre_s[b] - ssum[b]
    return below_s, b, m - below_c


def _frac_part(cnt_b, sum_b, r):
    safe = jnp.maximum(cnt_b, 1.0)
    return jnp.where(cnt_b > 0.0, (r / safe) * sum_b, 0.0)


def kernel(data, labels):
    d = data.reshape(-1)
    lab = labels.reshape(-1)

    hist1 = _pass1(d, lab)
    h1 = hist1.reshape(NW, 4, NB, L).sum(axis=(0, 3))
    pc, ps, nc, ns = h1[0], h1[1], h1[2], h1[3]
    n_pos = jnp.sum(pc)
    n_neg = jnp.sum(nc)
    m = jnp.minimum(n_pos, n_neg)

    ps_above, bp, need_p = _select_desc(pc, ps, m)
    ns_below, bn, need_n = _select_asc(nc, ns, m)

    thr = jnp.concatenate(
        [jnp.full((L,), bp, jnp.int32), jnp.full((L,), bn, jnp.int32)]
    )
    hist2 = _pass2(d, lab, thr)
    h2 = hist2.reshape(NW, 4, NB, L).sum(axis=(0, 3))
    pc2, ps2, nc2, ns2 = h2[0], h2[1], h2[2], h2[3]

    ps2_above, sp, r_p = _select_desc(pc2, ps2, need_p)
    ns2_below, sn, r_n = _select_asc(nc2, ns2, need_n)

    sum_pos = ps_above + ps2_above + _frac_part(pc2[sp], ps2[sp], r_p)
    sum_neg = ns_below + ns2_below + _frac_part(nc2[sn], ns2[sn], r_n)

    mined = (sum_pos + sum_neg) / (2.0 * m)
    loss_all = (jnp.sum(ps) + jnp.sum(ns)) / jnp.float32(N_TOTAL)
    return jnp.where(n_neg == 0, loss_all, mined)
